# baseline (speedup 1.0000x reference)
import functools

import jax
import jax.numpy as jnp
from jax import lax
from jax.experimental import pallas as pl
from jax.experimental.pallas import tpu as pltpu
from jax.experimental.pallas import tpu_sc as plsc


def _dist_kernel(xt_ref, xf_ref, score_ref, *, n):
    xt = xt_ref[0]
    xf = xf_ref[0]
    sq_row = jnp.sum(xf * xf, axis=0, keepdims=True)
    sq_col = jnp.sum(xt * xt, axis=1, keepdims=True)
    inner = -2.0 * jax.lax.dot_general(
        xt, xf, (((1,), (0,)), ((), ())),
        preferred_element_type=jnp.float32,
        precision=lax.Precision.DEFAULT)
    score_ref[0] = (sq_col + inner) + sq_row


def _make_sc_topk_gather(n_rows, n, c, cp, k_nn, n_per_batch):
    num_cores, num_subcores = 2, 16
    nw = num_cores * num_subcores
    rows_w = n_rows // nw
    CH = 8
    mesh = plsc.VectorSubcoreMesh(core_axis_name="c", subcore_axis_name="s",
                                  num_cores=num_cores,
                                  num_subcores=num_subcores)

    @functools.partial(
        pl.kernel, mesh=mesh,
        out_type=jax.ShapeDtypeStruct((n_rows, cp), jnp.float32),
        compiler_params=pltpu.CompilerParams(needs_layout_passes=False),
        scratch_types=[
            pltpu.VMEM((CH, n), jnp.float32),
            pltpu.VMEM((CH, cp), jnp.float32),
            pltpu.VMEM((CH, k_nn, cp), jnp.float32),
            pltpu.VMEM((CH, 16), jnp.int32),
            pltpu.SemaphoreType.DMA,
            pltpu.SemaphoreType.DMA,
        ],
    )
    def sc_kernel(score_hbm, xtf_hbm, out_hbm, row_v, out_v, gbuf, idx_v,
                  sem_row, sem_g):
        wid = lax.axis_index("s") * num_cores + lax.axis_index("c")
        base = wid * rows_w
        goff = (base // n_per_batch) * n_per_batch
        iota16 = lax.iota(jnp.int32, 16)

        def tree_topk(r, lo, hi, descending):
            if hi - lo == 1:
                v = row_v[r, pl.ds(lo * 16, 16)]
                return plsc.sort_key_val(v, iota16 + lo * 16,
                                         descending=descending)
            mid = (lo + hi) // 2
            lv, li = tree_topk(r, lo, mid, False)
            rv, ri = tree_topk(r, mid, hi, True)
            take = rv < lv
            mv = jnp.where(take, rv, lv)
            mi = jnp.where(take, ri, li)
            if lo == 0 and hi == n // 16:
                return mv, mi
            return plsc.sort_key_val(mv, mi, descending=descending)

        def chunk_body(g, _):
            row0 = base + g * CH
            pltpu.sync_copy(score_hbm.at[pl.ds(row0, CH)], row_v)

            def row_body(r, carry):
                _, mi = tree_topk(r, 0, n // 16, False)
                idx_v[r] = mi + goff
                return carry

            lax.fori_loop(0, CH, row_body, 0)

            copies = []
            for t in range(CH):
                copies.append(
                    pltpu.async_copy(xtf_hbm.at[idx_v.at[t]], gbuf.at[t],
                                     sem_g))
            for cpd in copies:
                cpd.wait()
            for t in range(CH):
                for cc in range(c // 16):
                    m = gbuf[t, 0, pl.ds(cc * 16, 16)]
                    for k in range(1, k_nn):
                        m = jnp.maximum(m, gbuf[t, k, pl.ds(cc * 16, 16)])
                    out_v[t, pl.ds(cc * 16, 16)] = m
            pltpu.sync_copy(out_v, out_hbm.at[pl.ds(row0, CH)])
            return 0

        lax.fori_loop(0, rows_w // CH, chunk_body, 0)

    return sc_kernel


def _conv_stats_kernel(xt_ref, xjm_ref, wet_ref, wot_ref, bc_ref,
                       y_ref, sums_ref, sumsq_ref):
    b = pl.program_id(0)
    xt = xt_ref[0]
    xjmax = xjm_ref[0] - xt
    y = (jax.lax.dot_general(xt, wet_ref[...], (((1,), (0,)), ((), ())),
                             preferred_element_type=jnp.float32,
                             precision=lax.Precision.DEFAULT)
         + jax.lax.dot_general(xjmax, wot_ref[...], (((1,), (0,)), ((), ())),
                               preferred_element_type=jnp.float32,
                               precision=lax.Precision.DEFAULT)
         + bc_ref[...])
    y_ref[0] = y
    part_s = jnp.sum(y, axis=0, keepdims=True)
    part_q = jnp.sum(y * y, axis=0, keepdims=True)

    @pl.when(b == 0)
    def _():
        sums_ref[...] = part_s
        sumsq_ref[...] = part_q

    @pl.when(b > 0)
    def _():
        sums_ref[...] = sums_ref[...] + part_s
        sumsq_ref[...] = sumsq_ref[...] + part_q


def _bn_relu_kernel(y_ref, sums_ref, sumsq_ref, gamma_ref, beta_ref,
                    out_ref, *, count):
    mean = sums_ref[...] / count
    var = sumsq_ref[...] / count - mean * mean
    inv = 1.0 / jnp.sqrt(var + 1e-5)
    scale = gamma_ref[...] * inv
    shift = beta_ref[...] - mean * scale
    out_ref[0] = jnp.maximum(y_ref[0] * scale + shift, 0.0)


def kernel(x, Wc, bc, gamma, beta):
    b, c, h, w = x.shape
    n = h * w
    o = Wc.shape[0]
    k_nn = 16

    xf = x.reshape(b, c, n)
    xt = jnp.transpose(xf, (0, 2, 1))
    wet = jnp.transpose(Wc[:, 0::2], (1, 0))
    wot = jnp.transpose(Wc[:, 1::2], (1, 0))
    bc2 = bc.reshape(1, o)

    score = pl.pallas_call(
        functools.partial(_dist_kernel, n=n),
        grid=(b,),
        in_specs=[
            pl.BlockSpec((1, n, c), lambda i: (i, 0, 0)),
            pl.BlockSpec((1, c, n), lambda i: (i, 0, 0)),
        ],
        out_specs=pl.BlockSpec((1, n, n), lambda i: (i, 0, 0)),
        out_shape=jax.ShapeDtypeStruct((b, n, n), jnp.float32),
    )(xt, xf)

    cp = 128
    score2 = score.reshape(b * n, n)
    xtf = jnp.pad(xt.reshape(b * n, c), ((0, 0), (0, cp - c)))
    sc_k = _make_sc_topk_gather(b * n, n, c, cp, k_nn, n)
    xjm_raw = sc_k(score2, xtf)
    xjm = xjm_raw[:, :c].reshape(b, n, c)

    y_raw, sums, sumsq = pl.pallas_call(
        _conv_stats_kernel,
        grid=(b,),
        in_specs=[
            pl.BlockSpec((1, n, c), lambda i: (i, 0, 0)),
            pl.BlockSpec((1, n, c), lambda i: (i, 0, 0)),
            pl.BlockSpec((c, o), lambda i: (0, 0)),
            pl.BlockSpec((c, o), lambda i: (0, 0)),
            pl.BlockSpec((1, o), lambda i: (0, 0)),
        ],
        out_specs=[
            pl.BlockSpec((1, n, o), lambda i: (i, 0, 0)),
            pl.BlockSpec((1, o), lambda i: (0, 0)),
            pl.BlockSpec((1, o), lambda i: (0, 0)),
        ],
        out_shape=[
            jax.ShapeDtypeStruct((b, n, o), jnp.float32),
            jax.ShapeDtypeStruct((1, o), jnp.float32),
            jax.ShapeDtypeStruct((1, o), jnp.float32),
        ],
    )(xt, xjm, wet, wot, bc2)

    out = pl.pallas_call(
        functools.partial(_bn_relu_kernel, count=float(b * n)),
        grid=(b,),
        in_specs=[
            pl.BlockSpec((1, n, o), lambda i: (i, 0, 0)),
            pl.BlockSpec((1, o), lambda i: (0, 0)),
            pl.BlockSpec((1, o), lambda i: (0, 0)),
            pl.BlockSpec((1, o), lambda i: (0, 0)),
            pl.BlockSpec((1, o), lambda i: (0, 0)),
        ],
        out_specs=pl.BlockSpec((1, n, o), lambda i: (i, 0, 0)),
        out_shape=jax.ShapeDtypeStruct((b, n, o), jnp.float32),
    )(y_raw, sums, sumsq, gamma.reshape(1, o), beta.reshape(1, o))

    return jnp.transpose(out, (0, 2, 1)).reshape(b, o, h, w)


# speedup vs baseline: 1.2497x; 1.2497x over previous
import functools

import jax
import jax.numpy as jnp
from jax import lax
from jax.experimental import pallas as pl
from jax.experimental.pallas import tpu as pltpu
from jax.experimental.pallas import tpu_sc as plsc


def _dist_kernel(xt_ref, xf_ref, score_ref, *, n):
    xt = xt_ref[0]
    xf = xf_ref[0]
    sq_row = jnp.sum(xf * xf, axis=0, keepdims=True)
    sq_col = jnp.sum(xt * xt, axis=1, keepdims=True)
    inner = -2.0 * jax.lax.dot_general(
        xt, xf, (((1,), (0,)), ((), ())),
        preferred_element_type=jnp.float32,
        precision=lax.Precision.DEFAULT)
    score_ref[0] = (sq_col + inner) + sq_row


def _make_sc_topk_gather(n_rows, n, c, cp, k_nn, n_per_batch):
    num_cores, num_subcores = 2, 16
    nw = num_cores * num_subcores
    rows_w = n_rows // nw
    CH = 8
    mesh = plsc.VectorSubcoreMesh(core_axis_name="c", subcore_axis_name="s",
                                  num_cores=num_cores,
                                  num_subcores=num_subcores)

    @functools.partial(
        pl.kernel, mesh=mesh,
        out_type=jax.ShapeDtypeStruct((n_rows, cp), jnp.float32),
        compiler_params=pltpu.CompilerParams(needs_layout_passes=False),
        scratch_types=[
            pltpu.VMEM((CH, n), jnp.float32),
            pltpu.VMEM((CH, cp), jnp.float32),
            pltpu.VMEM((CH, k_nn, cp), jnp.float32),
            pltpu.SemaphoreType.DMA,
            pltpu.SemaphoreType.DMA,
        ],
    )
    def sc_kernel(score_hbm, xtf_hbm, out_hbm, row_v, out_v, gbuf,
                  sem_row, sem_g):
        wid = lax.axis_index("s") * num_cores + lax.axis_index("c")
        base = wid * rows_w
        goff = (base // n_per_batch) * n_per_batch
        iota16 = lax.iota(jnp.int32, 16)

        def chunk_body(t0, _):
            row0 = base + t0 * CH
            pltpu.sync_copy(score_hbm.at[pl.ds(row0, CH)], row_v)

            init = []
            for t in range(CH):
                init.extend([jnp.full((16,), jnp.inf, jnp.float32),
                             jnp.zeros((16,), jnp.int32)])

            @plsc.parallel_loop(0, n // 16, unroll=2, carry=tuple(init))
            def res(blk, carry):
                new = []
                for t in range(CH):
                    cv, ci = carry[2 * t], carry[2 * t + 1]
                    v = row_v[t, pl.ds(blk * 16, 16)]
                    dv, di = plsc.sort_key_val(v, iota16 + blk * 16,
                                               descending=True)
                    take = dv < cv
                    mv = jnp.where(take, dv, cv)
                    mi = jnp.where(take, di, ci)
                    sv, si = plsc.sort_key_val(mv, mi)
                    new.extend([sv, si])
                return tuple(new)

            copies = []
            for t in range(CH):
                gidx = res[2 * t + 1] + goff
                copies.append(
                    pltpu.async_copy(xtf_hbm.at[gidx], gbuf.at[t], sem_g))
            for cpd in copies:
                cpd.wait()
            for t in range(CH):
                for cc in range(c // 16):
                    m = gbuf[t, 0, pl.ds(cc * 16, 16)]
                    for k in range(1, k_nn):
                        m = jnp.maximum(m, gbuf[t, k, pl.ds(cc * 16, 16)])
                    out_v[t, pl.ds(cc * 16, 16)] = m
            pltpu.sync_copy(out_v, out_hbm.at[pl.ds(row0, CH)])
            return 0

        lax.fori_loop(0, rows_w // CH, chunk_body, 0)

    return sc_kernel


def _conv_stats_kernel(xt_ref, xjm_ref, wet_ref, wot_ref, bc_ref,
                       y_ref, sums_ref, sumsq_ref):
    b = pl.program_id(0)
    xt = xt_ref[0]
    xjmax = xjm_ref[0] - xt
    y = (jax.lax.dot_general(xt, wet_ref[...], (((1,), (0,)), ((), ())),
                             preferred_element_type=jnp.float32,
                             precision=lax.Precision.DEFAULT)
         + jax.lax.dot_general(xjmax, wot_ref[...], (((1,), (0,)), ((), ())),
                               preferred_element_type=jnp.float32,
                               precision=lax.Precision.DEFAULT)
         + bc_ref[...])
    y_ref[0] = y
    part_s = jnp.sum(y, axis=0, keepdims=True)
    part_q = jnp.sum(y * y, axis=0, keepdims=True)

    @pl.when(b == 0)
    def _():
        sums_ref[...] = part_s
        sumsq_ref[...] = part_q

    @pl.when(b > 0)
    def _():
        sums_ref[...] = sums_ref[...] + part_s
        sumsq_ref[...] = sumsq_ref[...] + part_q


def _bn_relu_kernel(y_ref, sums_ref, sumsq_ref, gamma_ref, beta_ref,
                    out_ref, *, count):
    mean = sums_ref[...] / count
    var = sumsq_ref[...] / count - mean * mean
    inv = 1.0 / jnp.sqrt(var + 1e-5)
    scale = gamma_ref[...] * inv
    shift = beta_ref[...] - mean * scale
    out_ref[0] = jnp.maximum(y_ref[0] * scale + shift, 0.0)


def kernel(x, Wc, bc, gamma, beta):
    b, c, h, w = x.shape
    n = h * w
    o = Wc.shape[0]
    k_nn = 16

    xf = x.reshape(b, c, n)
    xt = jnp.transpose(xf, (0, 2, 1))
    wet = jnp.transpose(Wc[:, 0::2], (1, 0))
    wot = jnp.transpose(Wc[:, 1::2], (1, 0))
    bc2 = bc.reshape(1, o)

    score = pl.pallas_call(
        functools.partial(_dist_kernel, n=n),
        grid=(b,),
        in_specs=[
            pl.BlockSpec((1, n, c), lambda i: (i, 0, 0)),
            pl.BlockSpec((1, c, n), lambda i: (i, 0, 0)),
        ],
        out_specs=pl.BlockSpec((1, n, n), lambda i: (i, 0, 0)),
        out_shape=jax.ShapeDtypeStruct((b, n, n), jnp.float32),
    )(xt, xf)

    cp = 128
    score2 = score.reshape(b * n, n)
    xtf = jnp.pad(xt.reshape(b * n, c), ((0, 0), (0, cp - c)))
    sc_k = _make_sc_topk_gather(b * n, n, c, cp, k_nn, n)
    xjm_raw = sc_k(score2, xtf)
    xjm = xjm_raw[:, :c].reshape(b, n, c)

    y_raw, sums, sumsq = pl.pallas_call(
        _conv_stats_kernel,
        grid=(b,),
        in_specs=[
            pl.BlockSpec((1, n, c), lambda i: (i, 0, 0)),
            pl.BlockSpec((1, n, c), lambda i: (i, 0, 0)),
            pl.BlockSpec((c, o), lambda i: (0, 0)),
            pl.BlockSpec((c, o), lambda i: (0, 0)),
            pl.BlockSpec((1, o), lambda i: (0, 0)),
        ],
        out_specs=[
            pl.BlockSpec((1, n, o), lambda i: (i, 0, 0)),
            pl.BlockSpec((1, o), lambda i: (0, 0)),
            pl.BlockSpec((1, o), lambda i: (0, 0)),
        ],
        out_shape=[
            jax.ShapeDtypeStruct((b, n, o), jnp.float32),
            jax.ShapeDtypeStruct((1, o), jnp.float32),
            jax.ShapeDtypeStruct((1, o), jnp.float32),
        ],
    )(xt, xjm, wet, wot, bc2)

    out = pl.pallas_call(
        functools.partial(_bn_relu_kernel, count=float(b * n)),
        grid=(b,),
        in_specs=[
            pl.BlockSpec((1, n, o), lambda i: (i, 0, 0)),
            pl.BlockSpec((1, o), lambda i: (0, 0)),
            pl.BlockSpec((1, o), lambda i: (0, 0)),
            pl.BlockSpec((1, o), lambda i: (0, 0)),
            pl.BlockSpec((1, o), lambda i: (0, 0)),
        ],
        out_specs=pl.BlockSpec((1, n, o), lambda i: (i, 0, 0)),
        out_shape=jax.ShapeDtypeStruct((b, n, o), jnp.float32),
    )(y_raw, sums, sumsq, gamma.reshape(1, o), beta.reshape(1, o))

    return jnp.transpose(out, (0, 2, 1)).reshape(b, o, h, w)


# speedup vs baseline: 1.8103x; 1.4487x over previous
import functools

import jax
import jax.numpy as jnp
from jax import lax
from jax.experimental import pallas as pl
from jax.experimental.pallas import tpu as pltpu
from jax.experimental.pallas import tpu_sc as plsc


def _dist_kernel(xt_ref, xf_ref, score_ref, *, n):
    xt = xt_ref[0]
    xf = xf_ref[0]
    sq_row = jnp.sum(xf * xf, axis=0, keepdims=True)
    sq_col = jnp.sum(xt * xt, axis=1, keepdims=True)
    inner = -2.0 * jax.lax.dot_general(
        xt, xf, (((1,), (0,)), ((), ())),
        preferred_element_type=jnp.float32,
        precision=lax.Precision.DEFAULT)
    score_ref[0] = (sq_col + inner) + sq_row


def _make_sc_topk_gather(n_rows, row_start, n, c, cp, k_nn, n_per_batch):
    num_cores, num_subcores = 2, 16
    nw = num_cores * num_subcores
    rows_w = (n_rows - row_start) // nw
    CH = 8
    mesh = plsc.VectorSubcoreMesh(core_axis_name="c", subcore_axis_name="s",
                                  num_cores=num_cores,
                                  num_subcores=num_subcores)

    @functools.partial(
        pl.kernel, mesh=mesh,
        out_type=jax.ShapeDtypeStruct((n_rows - row_start, cp), jnp.float32),
        compiler_params=pltpu.CompilerParams(needs_layout_passes=False),
        scratch_types=[
            pltpu.VMEM((CH, n), jnp.float32),
            pltpu.VMEM((CH, cp), jnp.float32),
            pltpu.VMEM((CH, k_nn, cp), jnp.float32),
            pltpu.SemaphoreType.DMA,
            pltpu.SemaphoreType.DMA,
        ],
    )
    def sc_kernel(score_hbm, xtf_hbm, out_hbm, row_v, out_v, gbuf,
                  sem_row, sem_g):
        wid = lax.axis_index("s") * num_cores + lax.axis_index("c")
        base = row_start + wid * rows_w
        goff = (base // n_per_batch) * n_per_batch
        iota16 = lax.iota(jnp.int32, 16)

        def chunk_body(t0, _):
            row0 = base + t0 * CH
            pltpu.sync_copy(score_hbm.at[pl.ds(row0, CH)], row_v)

            def blk_body(blk, carry):
                new = []
                for t in range(CH):
                    cv, ci = carry[2 * t], carry[2 * t + 1]
                    v = row_v[t, pl.ds(blk * 16, 16)]
                    dv, di = plsc.sort_key_val(v, iota16 + blk * 16,
                                               descending=True)
                    take = dv < cv
                    mv = jnp.where(take, dv, cv)
                    mi = jnp.where(take, di, ci)
                    sv, si = plsc.sort_key_val(mv, mi)
                    new.extend([sv, si])
                return tuple(new)

            init = []
            for t in range(CH):
                init.extend([jnp.full((16,), jnp.inf, jnp.float32),
                             jnp.zeros((16,), jnp.int32)])
            res = lax.fori_loop(0, n // 16, blk_body, tuple(init))

            copies = []
            for t in range(CH):
                gidx = res[2 * t + 1] + goff
                copies.append(
                    pltpu.async_copy(xtf_hbm.at[gidx], gbuf.at[t], sem_g))
            for cpd in copies:
                cpd.wait()
            for t in range(CH):
                for cc in range(c // 16):
                    m = gbuf[t, 0, pl.ds(cc * 16, 16)]
                    for k in range(1, k_nn):
                        m = jnp.maximum(m, gbuf[t, k, pl.ds(cc * 16, 16)])
                    out_v[t, pl.ds(cc * 16, 16)] = m
            pltpu.sync_copy(out_v, out_hbm.at[pl.ds(row0 - row_start, CH)])
            return 0

        lax.fori_loop(0, rows_w // CH, chunk_body, 0)

    return sc_kernel


def _tc_topk_kernel(score_ref, xt_ref, xjm_ref, score_scr, *, n, c, k_nn):
    xt = xt_ref[0]
    score_scr[...] = score_ref[0]
    jidx = lax.broadcasted_iota(jnp.int32, (n, n), 1)

    def body(_, carry):
        s = score_scr[...]
        rowmin = jnp.min(s, axis=1, keepdims=True)
        cand = jnp.where(s <= rowmin, jidx, n)
        minidx = jnp.min(cand, axis=1, keepdims=True)
        sel = jidx == minidx
        onehot = sel.astype(jnp.float32)
        score_scr[...] = jnp.where(sel, jnp.inf, s)
        g = jax.lax.dot_general(
            onehot, xt, (((1,), (0,)), ((), ())),
            preferred_element_type=jnp.float32,
            precision=lax.Precision.DEFAULT)
        return jnp.maximum(carry, g)

    runmax = lax.fori_loop(
        0, k_nn, body, jnp.full((n, c), -jnp.inf, dtype=jnp.float32))
    xjm_ref[0] = runmax


def _conv_stats_kernel(xt_ref, xjm_ref, wet_ref, wot_ref, bc_ref,
                       y_ref, sums_ref, sumsq_ref):
    b = pl.program_id(0)
    xt = xt_ref[0]
    xjmax = xjm_ref[0] - xt
    y = (jax.lax.dot_general(xt, wet_ref[...], (((1,), (0,)), ((), ())),
                             preferred_element_type=jnp.float32,
                             precision=lax.Precision.DEFAULT)
         + jax.lax.dot_general(xjmax, wot_ref[...], (((1,), (0,)), ((), ())),
                               preferred_element_type=jnp.float32,
                               precision=lax.Precision.DEFAULT)
         + bc_ref[...])
    y_ref[0] = y
    part_s = jnp.sum(y, axis=0, keepdims=True)
    part_q = jnp.sum(y * y, axis=0, keepdims=True)

    @pl.when(b == 0)
    def _():
        sums_ref[...] = part_s
        sumsq_ref[...] = part_q

    @pl.when(b > 0)
    def _():
        sums_ref[...] = sums_ref[...] + part_s
        sumsq_ref[...] = sumsq_ref[...] + part_q


def _bn_relu_kernel(y_ref, sums_ref, sumsq_ref, gamma_ref, beta_ref,
                    out_ref, *, count):
    mean = sums_ref[...] / count
    var = sumsq_ref[...] / count - mean * mean
    inv = 1.0 / jnp.sqrt(var + 1e-5)
    scale = gamma_ref[...] * inv
    shift = beta_ref[...] - mean * scale
    out_ref[0] = jnp.maximum(y_ref[0] * scale + shift, 0.0)


def kernel(x, Wc, bc, gamma, beta):
    b, c, h, w = x.shape
    n = h * w
    o = Wc.shape[0]
    k_nn = 16

    xf = x.reshape(b, c, n)
    xt = jnp.transpose(xf, (0, 2, 1))
    wet = jnp.transpose(Wc[:, 0::2], (1, 0))
    wot = jnp.transpose(Wc[:, 1::2], (1, 0))
    bc2 = bc.reshape(1, o)

    score = pl.pallas_call(
        functools.partial(_dist_kernel, n=n),
        grid=(b,),
        in_specs=[
            pl.BlockSpec((1, n, c), lambda i: (i, 0, 0)),
            pl.BlockSpec((1, c, n), lambda i: (i, 0, 0)),
        ],
        out_specs=pl.BlockSpec((1, n, n), lambda i: (i, 0, 0)),
        out_shape=jax.ShapeDtypeStruct((b, n, n), jnp.float32),
    )(xt, xf)

    cp = 128
    split = b // 2
    score2 = score.reshape(b * n, n)
    xtf = jnp.pad(xt.reshape(b * n, c), ((0, 0), (0, cp - c)))
    sc_k = _make_sc_topk_gather(b * n, split * n, n, c, cp, k_nn, n)
    xjm_sc = sc_k(score2, xtf)

    xjm_tc = pl.pallas_call(
        functools.partial(_tc_topk_kernel, n=n, c=c, k_nn=k_nn),
        grid=(split,),
        in_specs=[
            pl.BlockSpec((1, n, n), lambda i: (i, 0, 0)),
            pl.BlockSpec((1, n, c), lambda i: (i, 0, 0)),
        ],
        out_specs=pl.BlockSpec((1, n, c), lambda i: (i, 0, 0)),
        out_shape=jax.ShapeDtypeStruct((split, n, c), jnp.float32),
        scratch_shapes=[pltpu.VMEM((n, n), jnp.float32)],
    )(score, xt)

    xjm = jnp.concatenate(
        [xjm_tc, xjm_sc[:, :c].reshape(b - split, n, c)], axis=0)

    y_raw, sums, sumsq = pl.pallas_call(
        _conv_stats_kernel,
        grid=(b,),
        in_specs=[
            pl.BlockSpec((1, n, c), lambda i: (i, 0, 0)),
            pl.BlockSpec((1, n, c), lambda i: (i, 0, 0)),
            pl.BlockSpec((c, o), lambda i: (0, 0)),
            pl.BlockSpec((c, o), lambda i: (0, 0)),
            pl.BlockSpec((1, o), lambda i: (0, 0)),
        ],
        out_specs=[
            pl.BlockSpec((1, n, o), lambda i: (i, 0, 0)),
            pl.BlockSpec((1, o), lambda i: (0, 0)),
            pl.BlockSpec((1, o), lambda i: (0, 0)),
        ],
        out_shape=[
            jax.ShapeDtypeStruct((b, n, o), jnp.float32),
            jax.ShapeDtypeStruct((1, o), jnp.float32),
            jax.ShapeDtypeStruct((1, o), jnp.float32),
        ],
    )(xt, xjm, wet, wot, bc2)

    out = pl.pallas_call(
        functools.partial(_bn_relu_kernel, count=float(b * n)),
        grid=(b,),
        in_specs=[
            pl.BlockSpec((1, n, o), lambda i: (i, 0, 0)),
            pl.BlockSpec((1, o), lambda i: (0, 0)),
            pl.BlockSpec((1, o), lambda i: (0, 0)),
            pl.BlockSpec((1, o), lambda i: (0, 0)),
            pl.BlockSpec((1, o), lambda i: (0, 0)),
        ],
        out_specs=pl.BlockSpec((1, n, o), lambda i: (i, 0, 0)),
        out_shape=jax.ShapeDtypeStruct((b, n, o), jnp.float32),
    )(y_raw, sums, sumsq, gamma.reshape(1, o), beta.reshape(1, o))

    return jnp.transpose(out, (0, 2, 1)).reshape(b, o, h, w)
